# Initial kernel scaffold; baseline (speedup 1.0000x reference)
import jax
import jax.numpy as jnp
from jax.experimental import pallas as pl


def kernel(edge_index, batch_idx, stance_features, relu_output, conv_weight, mul_matrix_1, bias_matrix_2, W3, b3, W4, b4):
    raise NotImplementedError("write your pallas kernel here")


# speedup vs baseline: 3.0464x; 3.0464x over previous
import functools

import jax
import jax.numpy as jnp
from jax import lax
from jax.experimental import pallas as pl
from jax.experimental.pallas import tpu as pltpu

N = 100000
G = 512
NB = 2000
NBLK = N // NB


def _pool_body(batch_ref, stance_ref, relu_ref, acc_ref, a_ref, b_ref):
    i = pl.program_id(0)
    b = batch_ref[0, 0, :]
    gids = lax.broadcasted_iota(jnp.int32, (G, NB), 0)
    oh_f = jnp.where(gids == b[None, :], 1.0, 0.0).astype(jnp.float32)
    oh_bf = oh_f.astype(jnp.bfloat16)
    x = jnp.concatenate(
        [stance_ref[...].astype(jnp.bfloat16), relu_ref[...].astype(jnp.bfloat16)],
        axis=1,
    )
    part_a = jnp.dot(oh_bf, x, preferred_element_type=jnp.float32)
    part_b4 = jnp.dot(oh_f, acc_ref[...], preferred_element_type=jnp.float32)
    cnt = jnp.sum(oh_f, axis=1)
    part_b = jnp.concatenate(
        [part_b4, cnt[:, None], jnp.zeros((G, 3), jnp.float32)], axis=1
    )

    @pl.when(i == 0)
    def _():
        a_ref[...] = jnp.zeros_like(a_ref)
        b_ref[...] = jnp.zeros_like(b_ref)

    a_ref[...] += part_a
    b_ref[...] += part_b


def _pool(batch3, stance, relu, acc):
    return pl.pallas_call(
        _pool_body,
        grid=(NBLK,),
        in_specs=[
            pl.BlockSpec((1, 1, NB), lambda i: (i, 0, 0)),
            pl.BlockSpec((NB, 4), lambda i: (i, 0)),
            pl.BlockSpec((NB, 124), lambda i: (i, 0)),
            pl.BlockSpec((NB, 4), lambda i: (i, 0)),
        ],
        out_specs=[
            pl.BlockSpec((G, 128), lambda i: (0, 0)),
            pl.BlockSpec((G, 8), lambda i: (0, 0)),
        ],
        out_shape=[
            jax.ShapeDtypeStruct((G, 128), jnp.float32),
            jax.ShapeDtypeStruct((G, 8), jnp.float32),
        ],
    )(batch3, stance, relu, acc)


def _finish_body(a_ref, b_ref, cw_ref, m1_ref, bias_ref, w3_ref, b3_ref,
                 w4_ref, b4_ref, out_ref):
    a = a_ref[...]
    bmat = b_ref[...]
    m = jnp.dot(m1_ref[...], cw_ref[...], preferred_element_type=jnp.float32)

    col128 = lax.broadcasted_iota(jnp.int32, (128, 4), 0)
    sel4 = jnp.where(col128 == lax.broadcasted_iota(jnp.int32, (128, 4), 1),
                     1.0, 0.0)
    col8 = lax.broadcasted_iota(jnp.int32, (8, 4), 0)
    sel8 = jnp.where(col8 == lax.broadcasted_iota(jnp.int32, (8, 4), 1), 1.0, 0.0)
    e5 = jnp.where(lax.broadcasted_iota(jnp.int32, (8, 1), 0) == 4, 1.0, 0.0)

    self4 = jnp.dot(a, sel4, preferred_element_type=jnp.float32)
    edge4 = jnp.dot(bmat, sel8, preferred_element_type=jnp.float32)
    cnt = jnp.dot(bmat, e5, preferred_element_type=jnp.float32)
    conv4 = jnp.dot(self4 + edge4, m, preferred_element_type=jnp.float32)
    patch4 = conv4 - self4 - cnt * bias_ref[...]
    pre = a + jnp.dot(patch4, sel4.T, preferred_element_type=jnp.float32)
    mean = pre / jnp.maximum(cnt, 1.0)

    h = lax.dot_general(mean, w3_ref[...], (((1,), (1,)), ((), ())),
                        preferred_element_type=jnp.float32) + b3_ref[...]
    h = jnp.maximum(h, 0.0)
    out = lax.dot_general(h, w4_ref[...], (((1,), (1,)), ((), ())),
                          preferred_element_type=jnp.float32) + b4_ref[...]
    out_ref[...] = out


def _finish(a, bmat, cw, m1, bias, w3, b3, w4, b4):
    return pl.pallas_call(
        _finish_body,
        out_shape=jax.ShapeDtypeStruct((G, 8), jnp.float32),
    )(a, bmat, cw, m1, bias, w3, b3.reshape(1, 256), w4, b4.reshape(1, 8))


def kernel(edge_index, batch_idx, stance_features, relu_output, conv_weight,
           mul_matrix_1, bias_matrix_2, W3, b3, W4, b4):
    acc = jax.ops.segment_sum(stance_features[edge_index[0]], edge_index[1],
                              num_segments=N)
    batch3 = batch_idx.reshape(NBLK, 1, NB)
    a, bmat = _pool(batch3, stance_features, relu_output, acc)
    return _finish(a, bmat, conv_weight, mul_matrix_1, bias_matrix_2,
                   W3, b3, W4, b4)


# speedup vs baseline: 196.3576x; 64.4558x over previous
import functools

import jax
import jax.numpy as jnp
from jax import lax
from jax.experimental import pallas as pl
from jax.experimental.pallas import tpu as pltpu
from jax.experimental.pallas import tpu_sc as plsc

N = 100000
G = 512
NB = 2000
NBLK = N // NB

EDGES = 6400000
NC = 2
NS = 16
NWORK = NC * NS
CHUNK = 128
KCH = 16
STAGE = CHUNK * KCH
NSTAGES = EDGES // STAGE
BASE_STAGES = NSTAGES // NWORK
EXTRA = NSTAGES % NWORK
ROWS_PER_TILE = 6256
NPAD = ROWS_PER_TILE * NS


D8 = 8


def _sc_edge_body(edge4_hbm, stance_hbm, zeros_hbm, out_hbm,
                  row_v, col_v, gbufs, table_s, acc_s, gsem, ssem):
    c = lax.axis_index("c")
    s = lax.axis_index("s")
    wid = s * NC + c
    sl = pl.ds(s * ROWS_PER_TILE, ROWS_PER_TILE)
    pltpu.sync_copy(stance_hbm.at[sl], table_s.at[sl])
    pltpu.sync_copy(zeros_hbm.at[sl], acc_s.at[sl])
    plsc.subcore_barrier()

    nst = BASE_STAGES + jnp.where(wid < EXTRA, 1, 0)

    def stage_body(j, carry):
        sid = wid + j * NWORK
        pltpu.sync_copy(edge4_hbm.at[0, sid], row_v)
        pltpu.sync_copy(edge4_hbm.at[1, sid], col_v)
        gs = [pltpu.async_copy(table_s.at[row_v.at[k]], gbufs.at[k], gsem)
              for k in range(KCH)]
        for g in gs:
            g.wait()
        ss = [pltpu.async_copy(gbufs.at[k], acc_s.at[col_v.at[k]], ssem,
                               add=True)
              for k in range(KCH)]
        for t in ss:
            t.wait()
        return carry

    lax.fori_loop(0, nst, stage_body, 0)
    plsc.subcore_barrier()
    pltpu.sync_copy(acc_s.at[sl], out_hbm.at[c, sl])


def _sc_edge(edge_index, stance):
    edge4 = edge_index.reshape(2, NSTAGES, KCH, CHUNK)
    stance_pad = jnp.pad(stance, ((0, NPAD - N), (0, D8 - 4)))
    zeros = jnp.zeros((NPAD, D8), jnp.float32)
    mesh = plsc.VectorSubcoreMesh(core_axis_name="c", subcore_axis_name="s")
    f = functools.partial(
        pl.kernel,
        mesh=mesh,
        compiler_params=pltpu.CompilerParams(use_tc_tiling_on_sc=False),
        out_type=jax.ShapeDtypeStruct((NC, NPAD, D8), jnp.float32),
        scratch_types=[
            pltpu.VMEM((KCH, CHUNK), jnp.int32),
            pltpu.VMEM((KCH, CHUNK), jnp.int32),
            pltpu.VMEM((KCH, CHUNK, D8), jnp.float32),
            pltpu.VMEM_SHARED((NPAD, D8), jnp.float32),
            pltpu.VMEM_SHARED((NPAD, D8), jnp.float32),
            pltpu.SemaphoreType.DMA,
            pltpu.SemaphoreType.DMA,
        ],
    )(_sc_edge_body)
    return f(edge4, stance_pad, zeros)


def _pool_body(batch_ref, stance_ref, relu_ref, acc0_ref, acc1_ref, a_ref, b_ref):
    i = pl.program_id(0)
    b = batch_ref[0, 0, :]
    gids = lax.broadcasted_iota(jnp.int32, (G, NB), 0)
    oh_f = jnp.where(gids == b[None, :], 1.0, 0.0).astype(jnp.float32)
    oh_bf = oh_f.astype(jnp.bfloat16)
    x = jnp.concatenate(
        [stance_ref[...].astype(jnp.bfloat16), relu_ref[...].astype(jnp.bfloat16)],
        axis=1,
    )
    part_a = jnp.dot(oh_bf, x, preferred_element_type=jnp.float32)
    acc8 = acc0_ref[...] + acc1_ref[...]
    part_b4 = jnp.dot(oh_f, acc8[:, :4], preferred_element_type=jnp.float32)
    cnt = jnp.sum(oh_f, axis=1)
    part_b = jnp.concatenate(
        [part_b4, cnt[:, None], jnp.zeros((G, 3), jnp.float32)], axis=1
    )

    @pl.when(i == 0)
    def _():
        a_ref[...] = jnp.zeros_like(a_ref)
        b_ref[...] = jnp.zeros_like(b_ref)

    a_ref[...] += part_a
    b_ref[...] += part_b


def _pool(batch3, stance, relu, acc0, acc1):
    return pl.pallas_call(
        _pool_body,
        grid=(NBLK,),
        in_specs=[
            pl.BlockSpec((1, 1, NB), lambda i: (i, 0, 0)),
            pl.BlockSpec((NB, 4), lambda i: (i, 0)),
            pl.BlockSpec((NB, 124), lambda i: (i, 0)),
            pl.BlockSpec((NB, D8), lambda i: (i, 0)),
            pl.BlockSpec((NB, D8), lambda i: (i, 0)),
        ],
        out_specs=[
            pl.BlockSpec((G, 128), lambda i: (0, 0)),
            pl.BlockSpec((G, 8), lambda i: (0, 0)),
        ],
        out_shape=[
            jax.ShapeDtypeStruct((G, 128), jnp.float32),
            jax.ShapeDtypeStruct((G, 8), jnp.float32),
        ],
    )(batch3, stance, relu, acc0, acc1)


def _finish_body(a_ref, b_ref, cw_ref, m1_ref, bias_ref, w3_ref, b3_ref,
                 w4_ref, b4_ref, out_ref):
    a = a_ref[...]
    bmat = b_ref[...]
    m = jnp.dot(m1_ref[...], cw_ref[...], preferred_element_type=jnp.float32)

    col128 = lax.broadcasted_iota(jnp.int32, (128, 4), 0)
    sel4 = jnp.where(col128 == lax.broadcasted_iota(jnp.int32, (128, 4), 1),
                     1.0, 0.0)
    col8 = lax.broadcasted_iota(jnp.int32, (8, 4), 0)
    sel8 = jnp.where(col8 == lax.broadcasted_iota(jnp.int32, (8, 4), 1), 1.0, 0.0)
    e5 = jnp.where(lax.broadcasted_iota(jnp.int32, (8, 1), 0) == 4, 1.0, 0.0)

    self4 = jnp.dot(a, sel4, preferred_element_type=jnp.float32)
    edge4 = jnp.dot(bmat, sel8, preferred_element_type=jnp.float32)
    cnt = jnp.dot(bmat, e5, preferred_element_type=jnp.float32)
    conv4 = jnp.dot(self4 + edge4, m, preferred_element_type=jnp.float32)
    patch4 = conv4 - self4 - cnt * bias_ref[...]
    pre = a + jnp.dot(patch4, sel4.T, preferred_element_type=jnp.float32)
    mean = pre / jnp.maximum(cnt, 1.0)

    h = lax.dot_general(mean, w3_ref[...], (((1,), (1,)), ((), ())),
                        preferred_element_type=jnp.float32) + b3_ref[...]
    h = jnp.maximum(h, 0.0)
    out = lax.dot_general(h, w4_ref[...], (((1,), (1,)), ((), ())),
                          preferred_element_type=jnp.float32) + b4_ref[...]
    out_ref[...] = out


def _finish(a, bmat, cw, m1, bias, w3, b3, w4, b4):
    return pl.pallas_call(
        _finish_body,
        out_shape=jax.ShapeDtypeStruct((G, 8), jnp.float32),
    )(a, bmat, cw, m1, bias, w3, b3.reshape(1, 256), w4, b4.reshape(1, 8))


def kernel(edge_index, batch_idx, stance_features, relu_output, conv_weight,
           mul_matrix_1, bias_matrix_2, W3, b3, W4, b4):
    acc2 = _sc_edge(edge_index, stance_features)
    batch3 = batch_idx.reshape(NBLK, 1, NB)
    a, bmat = _pool(batch3, stance_features, relu_output,
                    acc2[0, :N], acc2[1, :N])
    return _finish(a, bmat, conv_weight, mul_matrix_1, bias_matrix_2,
                   W3, b3, W4, b4)


# speedup vs baseline: 198.9954x; 1.0134x over previous
import functools

import jax
import jax.numpy as jnp
from jax import lax
from jax.experimental import pallas as pl
from jax.experimental.pallas import tpu as pltpu
from jax.experimental.pallas import tpu_sc as plsc

N = 100000
G = 512
NB = 2000
NBLK = N // NB

EDGES = 6400000
NC = 2
NS = 16
NWORK = NC * NS
CHUNK = 128
KCH = 16
STAGE = CHUNK * KCH
NSTAGES = EDGES // STAGE
BASE_STAGES = NSTAGES // NWORK
EXTRA = NSTAGES % NWORK
ROWS_PER_TILE = 6256
NPAD = ROWS_PER_TILE * NS


D8 = 8


def _widen_body(stance_ref, out_ref):
    out_ref[...] = jnp.concatenate(
        [stance_ref[...], jnp.zeros((ROWS_PER_TILE, D8 - 4), jnp.float32)],
        axis=1)


def _widen(stance):
    return pl.pallas_call(
        _widen_body,
        grid=(NS,),
        in_specs=[pl.BlockSpec((ROWS_PER_TILE, 4), lambda i: (i, 0))],
        out_specs=pl.BlockSpec((ROWS_PER_TILE, D8), lambda i: (i, 0)),
        out_shape=jax.ShapeDtypeStruct((NPAD, D8), jnp.float32),
    )(stance)


def _sc_edge_body(edge4_hbm, stance_hbm, zeros_hbm, out_hbm,
                  row_v, col_v, gbufs, table_s, acc_s, gsem, ssem):
    c = lax.axis_index("c")
    s = lax.axis_index("s")
    wid = s * NC + c
    sl = pl.ds(s * ROWS_PER_TILE, ROWS_PER_TILE)
    pltpu.sync_copy(stance_hbm.at[sl], table_s.at[sl])
    pltpu.sync_copy(zeros_hbm, acc_s.at[sl])
    plsc.subcore_barrier()

    nst = BASE_STAGES + jnp.where(wid < EXTRA, 1, 0)

    def stage_body(j, carry):
        sid = wid + j * NWORK
        pltpu.sync_copy(edge4_hbm.at[0, sid], row_v)
        pltpu.sync_copy(edge4_hbm.at[1, sid], col_v)
        gs = [pltpu.async_copy(table_s.at[row_v.at[k]], gbufs.at[k], gsem)
              for k in range(KCH)]
        ss = []
        for k in range(KCH):
            gs[k].wait()
            ss.append(pltpu.async_copy(gbufs.at[k], acc_s.at[col_v.at[k]],
                                       ssem, add=True))
        for t in ss:
            t.wait()
        return carry

    lax.fori_loop(0, nst, stage_body, 0)
    plsc.subcore_barrier()
    pltpu.sync_copy(acc_s.at[sl], out_hbm.at[c, sl])


def _sc_edge(edge_index, stance):
    edge4 = edge_index.reshape(2, NSTAGES, KCH, CHUNK)
    zeros = jnp.zeros((ROWS_PER_TILE, D8), jnp.float32)
    mesh = plsc.VectorSubcoreMesh(core_axis_name="c", subcore_axis_name="s")
    f = functools.partial(
        pl.kernel,
        mesh=mesh,
        compiler_params=pltpu.CompilerParams(use_tc_tiling_on_sc=False),
        out_type=jax.ShapeDtypeStruct((NC, NPAD, D8), jnp.float32),
        scratch_types=[
            pltpu.VMEM((KCH, CHUNK), jnp.int32),
            pltpu.VMEM((KCH, CHUNK), jnp.int32),
            pltpu.VMEM((KCH, CHUNK, D8), jnp.float32),
            pltpu.VMEM_SHARED((NPAD, D8), jnp.float32),
            pltpu.VMEM_SHARED((NPAD, D8), jnp.float32),
            pltpu.SemaphoreType.DMA,
            pltpu.SemaphoreType.DMA,
        ],
    )(_sc_edge_body)
    return f(edge4, _widen(stance), zeros)


def _pool_body(batch_ref, stance_ref, relu_ref, acc0_ref, acc1_ref, a_ref, b_ref):
    i = pl.program_id(0)
    b = batch_ref[0, 0, :]
    gids = lax.broadcasted_iota(jnp.int32, (G, NB), 0)
    oh_f = jnp.where(gids == b[None, :], 1.0, 0.0).astype(jnp.float32)
    oh_bf = oh_f.astype(jnp.bfloat16)
    x = jnp.concatenate(
        [stance_ref[...].astype(jnp.bfloat16), relu_ref[...].astype(jnp.bfloat16)],
        axis=1,
    )
    part_a = jnp.dot(oh_bf, x, preferred_element_type=jnp.float32)
    acc8 = acc0_ref[...] + acc1_ref[...]
    part_b4 = jnp.dot(oh_f, acc8[:, :4], preferred_element_type=jnp.float32)
    cnt = jnp.sum(oh_f, axis=1)
    part_b = jnp.concatenate(
        [part_b4, cnt[:, None], jnp.zeros((G, 3), jnp.float32)], axis=1
    )

    @pl.when(i == 0)
    def _():
        a_ref[...] = jnp.zeros_like(a_ref)
        b_ref[...] = jnp.zeros_like(b_ref)

    a_ref[...] += part_a
    b_ref[...] += part_b


def _pool(batch3, stance, relu, acc0, acc1):
    return pl.pallas_call(
        _pool_body,
        grid=(NBLK,),
        in_specs=[
            pl.BlockSpec((1, 1, NB), lambda i: (i, 0, 0)),
            pl.BlockSpec((NB, 4), lambda i: (i, 0)),
            pl.BlockSpec((NB, 124), lambda i: (i, 0)),
            pl.BlockSpec((NB, D8), lambda i: (i, 0)),
            pl.BlockSpec((NB, D8), lambda i: (i, 0)),
        ],
        out_specs=[
            pl.BlockSpec((G, 128), lambda i: (0, 0)),
            pl.BlockSpec((G, 8), lambda i: (0, 0)),
        ],
        out_shape=[
            jax.ShapeDtypeStruct((G, 128), jnp.float32),
            jax.ShapeDtypeStruct((G, 8), jnp.float32),
        ],
    )(batch3, stance, relu, acc0, acc1)


def _finish_body(a_ref, b_ref, cw_ref, m1_ref, bias_ref, w3_ref, b3_ref,
                 w4_ref, b4_ref, out_ref):
    a = a_ref[...]
    bmat = b_ref[...]
    m = jnp.dot(m1_ref[...], cw_ref[...], preferred_element_type=jnp.float32)

    col128 = lax.broadcasted_iota(jnp.int32, (128, 4), 0)
    sel4 = jnp.where(col128 == lax.broadcasted_iota(jnp.int32, (128, 4), 1),
                     1.0, 0.0)
    col8 = lax.broadcasted_iota(jnp.int32, (8, 4), 0)
    sel8 = jnp.where(col8 == lax.broadcasted_iota(jnp.int32, (8, 4), 1), 1.0, 0.0)
    e5 = jnp.where(lax.broadcasted_iota(jnp.int32, (8, 1), 0) == 4, 1.0, 0.0)

    self4 = jnp.dot(a, sel4, preferred_element_type=jnp.float32)
    edge4 = jnp.dot(bmat, sel8, preferred_element_type=jnp.float32)
    cnt = jnp.dot(bmat, e5, preferred_element_type=jnp.float32)
    conv4 = jnp.dot(self4 + edge4, m, preferred_element_type=jnp.float32)
    patch4 = conv4 - self4 - cnt * bias_ref[...]
    pre = a + jnp.dot(patch4, sel4.T, preferred_element_type=jnp.float32)
    mean = pre / jnp.maximum(cnt, 1.0)

    h = lax.dot_general(mean, w3_ref[...], (((1,), (1,)), ((), ())),
                        preferred_element_type=jnp.float32) + b3_ref[...]
    h = jnp.maximum(h, 0.0)
    out = lax.dot_general(h, w4_ref[...], (((1,), (1,)), ((), ())),
                          preferred_element_type=jnp.float32) + b4_ref[...]
    out_ref[...] = out


def _finish(a, bmat, cw, m1, bias, w3, b3, w4, b4):
    return pl.pallas_call(
        _finish_body,
        out_shape=jax.ShapeDtypeStruct((G, 8), jnp.float32),
    )(a, bmat, cw, m1, bias, w3, b3.reshape(1, 256), w4, b4.reshape(1, 8))


def kernel(edge_index, batch_idx, stance_features, relu_output, conv_weight,
           mul_matrix_1, bias_matrix_2, W3, b3, W4, b4):
    acc2 = _sc_edge(edge_index, stance_features)
    batch3 = batch_idx.reshape(NBLK, 1, NB)
    a, bmat = _pool(batch3, stance_features, relu_output,
                    acc2[0, :N], acc2[1, :N])
    return _finish(a, bmat, conv_weight, mul_matrix_1, bias_matrix_2,
                   W3, b3, W4, b4)


# speedup vs baseline: 219.8903x; 1.1050x over previous
import functools

import jax
import jax.numpy as jnp
from jax import lax
from jax.experimental import pallas as pl
from jax.experimental.pallas import tpu as pltpu
from jax.experimental.pallas import tpu_sc as plsc

N = 100000
G = 512
NB = 2000
NBLK = N // NB

EDGES = 6400000
NC = 2
NS = 16
NWORK = NC * NS
CHUNK = 128
KCH = 16
STAGE = CHUNK * KCH
NSTAGES = EDGES // STAGE
BASE_STAGES = NSTAGES // NWORK
EXTRA = NSTAGES % NWORK
ROWS_PER_TILE = 6256
NPAD = ROWS_PER_TILE * NS


D8 = 8


def _widen_body(stance_ref, out_ref):
    out_ref[...] = jnp.concatenate(
        [stance_ref[...], jnp.zeros((ROWS_PER_TILE, D8 - 4), jnp.float32)],
        axis=1)


def _widen(stance):
    return pl.pallas_call(
        _widen_body,
        grid=(NS,),
        in_specs=[pl.BlockSpec((ROWS_PER_TILE, 4), lambda i: (i, 0))],
        out_specs=pl.BlockSpec((ROWS_PER_TILE, D8), lambda i: (i, 0)),
        out_shape=jax.ShapeDtypeStruct((NPAD, D8), jnp.float32),
    )(stance)


def _sc_edge_body(edge4_hbm, stance_hbm, zeros_hbm, out_hbm,
                  row_v, col_v, gbufs, table_s, acc_s, gsem, ssem):
    c = lax.axis_index("c")
    s = lax.axis_index("s")
    wid = s * NC + c
    sl = pl.ds(s * ROWS_PER_TILE, ROWS_PER_TILE)
    pltpu.sync_copy(stance_hbm.at[sl], table_s.at[sl])
    pltpu.sync_copy(zeros_hbm, acc_s.at[sl])
    plsc.subcore_barrier()

    nst = BASE_STAGES + jnp.where(wid < EXTRA, 1, 0)

    def stage_body(j, carry):
        sid = wid + j * NWORK
        pltpu.sync_copy(edge4_hbm.at[0, sid], row_v)
        pltpu.sync_copy(edge4_hbm.at[1, sid], col_v)
        gs = [pltpu.async_copy(table_s.at[row_v.at[k]], gbufs.at[k], gsem)
              for k in range(KCH)]
        ss = []
        for k in range(KCH):
            gs[k].wait()
            ss.append(pltpu.async_copy(gbufs.at[k], acc_s.at[col_v.at[k]],
                                       ssem, add=True))
        for t in ss:
            t.wait()
        return carry

    lax.fori_loop(0, nst, stage_body, 0)
    plsc.subcore_barrier()
    pltpu.sync_copy(acc_s.at[sl], out_hbm.at[c, sl])


def _sc_edge(edge_index, stance):
    edge4 = edge_index.reshape(2, NSTAGES, KCH, CHUNK)
    zeros = jnp.zeros((ROWS_PER_TILE, D8), jnp.float32)
    mesh = plsc.VectorSubcoreMesh(core_axis_name="c", subcore_axis_name="s")
    f = functools.partial(
        pl.kernel,
        mesh=mesh,
        compiler_params=pltpu.CompilerParams(use_tc_tiling_on_sc=False),
        out_type=jax.ShapeDtypeStruct((NC, NPAD, D8), jnp.float32),
        scratch_types=[
            pltpu.VMEM((KCH, CHUNK), jnp.int32),
            pltpu.VMEM((KCH, CHUNK), jnp.int32),
            pltpu.VMEM((KCH, CHUNK, D8), jnp.float32),
            pltpu.VMEM_SHARED((NPAD, D8), jnp.float32),
            pltpu.VMEM_SHARED((NPAD, D8), jnp.float32),
            pltpu.SemaphoreType.DMA,
            pltpu.SemaphoreType.DMA,
        ],
    )(_sc_edge_body)
    return f(edge4, _widen(stance), zeros)


def _pool_body(batch_ref, stance_ref, relu_ref, acc0_ref, acc1_ref, a_ref, b_ref):
    i = pl.program_id(0)
    b = batch_ref[0, 0, :]
    gids = lax.broadcasted_iota(jnp.int32, (G, NB), 0)
    oh_f = jnp.where(gids == b[None, :], 1.0, 0.0).astype(jnp.float32)
    oh_bf = oh_f.astype(jnp.bfloat16)
    x = jnp.concatenate(
        [stance_ref[...].astype(jnp.bfloat16), relu_ref[...].astype(jnp.bfloat16)],
        axis=1,
    )
    part_a = jnp.dot(oh_bf, x, preferred_element_type=jnp.float32)
    acc8 = acc0_ref[0] + acc1_ref[0]
    part_b4 = jnp.dot(oh_f, acc8[:, :4], preferred_element_type=jnp.float32)
    cnt = jnp.sum(oh_f, axis=1)
    part_b = jnp.concatenate(
        [part_b4, cnt[:, None], jnp.zeros((G, 3), jnp.float32)], axis=1
    )

    @pl.when(i == 0)
    def _():
        a_ref[...] = jnp.zeros_like(a_ref)
        b_ref[...] = jnp.zeros_like(b_ref)

    a_ref[...] += part_a
    b_ref[...] += part_b


def _pool(batch3, stance, relu, acc2):
    return pl.pallas_call(
        _pool_body,
        grid=(NBLK,),
        in_specs=[
            pl.BlockSpec((1, 1, NB), lambda i: (i, 0, 0)),
            pl.BlockSpec((NB, 4), lambda i: (i, 0)),
            pl.BlockSpec((NB, 124), lambda i: (i, 0)),
            pl.BlockSpec((1, NB, D8), lambda i: (0, i, 0)),
            pl.BlockSpec((1, NB, D8), lambda i: (1, i, 0)),
        ],
        out_specs=[
            pl.BlockSpec((G, 128), lambda i: (0, 0)),
            pl.BlockSpec((G, 8), lambda i: (0, 0)),
        ],
        out_shape=[
            jax.ShapeDtypeStruct((G, 128), jnp.float32),
            jax.ShapeDtypeStruct((G, 8), jnp.float32),
        ],
    )(batch3, stance, relu, acc2, acc2)


def _finish_body(a_ref, b_ref, cw_ref, m1_ref, bias_ref, w3_ref, b3_ref,
                 w4_ref, b4_ref, out_ref):
    a = a_ref[...]
    bmat = b_ref[...]
    m = jnp.dot(m1_ref[...], cw_ref[...], preferred_element_type=jnp.float32)

    col128 = lax.broadcasted_iota(jnp.int32, (128, 4), 0)
    sel4 = jnp.where(col128 == lax.broadcasted_iota(jnp.int32, (128, 4), 1),
                     1.0, 0.0)
    col8 = lax.broadcasted_iota(jnp.int32, (8, 4), 0)
    sel8 = jnp.where(col8 == lax.broadcasted_iota(jnp.int32, (8, 4), 1), 1.0, 0.0)
    e5 = jnp.where(lax.broadcasted_iota(jnp.int32, (8, 1), 0) == 4, 1.0, 0.0)

    self4 = jnp.dot(a, sel4, preferred_element_type=jnp.float32)
    edge4 = jnp.dot(bmat, sel8, preferred_element_type=jnp.float32)
    cnt = jnp.dot(bmat, e5, preferred_element_type=jnp.float32)
    conv4 = jnp.dot(self4 + edge4, m, preferred_element_type=jnp.float32)
    patch4 = conv4 - self4 - cnt * bias_ref[...]
    pre = a + jnp.dot(patch4, sel4.T, preferred_element_type=jnp.float32)
    mean = pre / jnp.maximum(cnt, 1.0)

    h = lax.dot_general(mean, w3_ref[...], (((1,), (1,)), ((), ())),
                        preferred_element_type=jnp.float32) + b3_ref[...]
    h = jnp.maximum(h, 0.0)
    out = lax.dot_general(h, w4_ref[...], (((1,), (1,)), ((), ())),
                          preferred_element_type=jnp.float32) + b4_ref[...]
    out_ref[...] = out


def _finish(a, bmat, cw, m1, bias, w3, b3, w4, b4):
    return pl.pallas_call(
        _finish_body,
        out_shape=jax.ShapeDtypeStruct((G, 8), jnp.float32),
    )(a, bmat, cw, m1, bias, w3, b3.reshape(1, 256), w4, b4.reshape(1, 8))


def kernel(edge_index, batch_idx, stance_features, relu_output, conv_weight,
           mul_matrix_1, bias_matrix_2, W3, b3, W4, b4):
    acc2 = _sc_edge(edge_index, stance_features)
    batch3 = batch_idx.reshape(NBLK, 1, NB)
    a, bmat = _pool(batch3, stance_features, relu_output, acc2)
    return _finish(a, bmat, conv_weight, mul_matrix_1, bias_matrix_2,
                   W3, b3, W4, b4)
